# baseline (speedup 1.0000x reference)
import jax
import jax.numpy as jnp
from jax.experimental import pallas as pl


_BLK = 1024


def _gather_kernel(grid_ref, x_ref, out_ref):
    g = grid_ref[0:1, :]
    idx = jnp.round((g + 1.0) * (512 - 1) / 2.0).astype(jnp.int32)
    idx = jnp.clip(idx, 0, 511)
    lane = jax.lax.broadcasted_iota(jnp.int32, (512, 16), 0)
    onehot = (lane == jnp.broadcast_to(idx, (512, 16))).astype(jnp.float32)
    out_ref[...] = jnp.dot(x_ref[...], onehot,
                           preferred_element_type=jnp.float32,
                           precision=jax.lax.Precision.HIGHEST)


def kernel(x, grid):
    b, c, r, w = x.shape
    n = b * c * r
    xf = x.reshape(n, w)
    gflat = jnp.broadcast_to(grid.reshape(1, -1), (8, grid.size))

    out = pl.pallas_call(
        _gather_kernel,
        grid=(n // _BLK,),
        in_specs=[
            pl.BlockSpec((8, 16), lambda i: (0, 0)),
            pl.BlockSpec((_BLK, w), lambda i: (i, 0)),
        ],
        out_specs=pl.BlockSpec((_BLK, 16), lambda i: (i, 0)),
        out_shape=jax.ShapeDtypeStruct((n, 16), jnp.float32),
    )(gflat, xf)

    return out.reshape(b, c, r, 8, 1, 2)
